# Initial kernel scaffold; baseline (speedup 1.0000x reference)
import jax
import jax.numpy as jnp
from jax.experimental import pallas as pl


def kernel(x, router_weight, w1, w2):
    raise NotImplementedError("write your pallas kernel here")


# speedup vs baseline: 1.8458x; 1.8458x over previous
import functools

import jax
import jax.numpy as jnp
from jax import lax
from jax.experimental import pallas as pl
from jax.experimental.pallas import tpu as pltpu
from jax.experimental.pallas import tpu_sc as plsc

T = 2048
H = 1024
F = 2048
E = 16
K = 2
CAP = (T * K // E) * 2

BC = 256
BF = 512
NF = F // BF
NB = 32
RT = (NB + 1) * BC
TRASH = NB * BC

NUM_SC_CORES = 2
NUM_SC_SUBCORES = 16
NW = NUM_SC_CORES * NUM_SC_SUBCORES
APW = T * K // NW
ACH = 32
TCH = ACH // K


def _router_body(x_ref, rw_ref, dst_ref, gw_ref, meta_ref):
    xf = x_ref[...]
    logits = jnp.dot(xf, rw_ref[...], preferred_element_type=jnp.float32)
    m = jnp.max(logits, axis=1, keepdims=True)
    p = jnp.exp(logits - m)
    p = p / jnp.sum(p, axis=1, keepdims=True)

    lane = lax.broadcasted_iota(jnp.int32, (T, E), 1)
    m1 = jnp.max(p, axis=1, keepdims=True)
    i1 = jnp.min(jnp.where(p == m1, lane, E), axis=1, keepdims=True)
    p2 = jnp.where(lane == i1, -1.0, p)
    m2 = jnp.max(p2, axis=1, keepdims=True)
    i2 = jnp.min(jnp.where(p2 == m2, lane, E), axis=1, keepdims=True)

    oh = (lane == i1).astype(jnp.float32) + (lane == i2).astype(jnp.float32)
    base = jnp.zeros((1, E), jnp.float32)
    chunks = []
    CH = 512
    ri = lax.broadcasted_iota(jnp.int32, (CH, CH), 0)
    ci = lax.broadcasted_iota(jnp.int32, (CH, CH), 1)
    tril = (ci < ri).astype(jnp.float32)
    for c in range(T // CH):
        ohc = oh[c * CH:(c + 1) * CH, :]
        excl = jnp.dot(tril, ohc, preferred_element_type=jnp.float32) + base
        chunks.append(excl)
        base = base + jnp.sum(ohc, axis=0, keepdims=True)
    posm = jnp.concatenate(chunks, axis=0)
    counts = base
    countc = jnp.minimum(counts, float(CAP))

    nb_e = jnp.ceil(countc / BC)
    lane16 = lax.broadcasted_iota(jnp.int32, (1, E), 1)
    ue = lax.broadcasted_iota(jnp.int32, (E, E), 0)
    ve = lax.broadcasted_iota(jnp.int32, (E, E), 1)
    umat = (ue < ve).astype(jnp.float32)
    blk_start = jnp.dot(nb_e, umat, preferred_element_type=jnp.float32)
    nbl = jnp.sum(nb_e)
    row_off = BC * blk_start

    pos1 = jnp.sum(jnp.where(lane == i1, posm, 0.0), axis=1, keepdims=True)
    pos2 = jnp.sum(jnp.where(lane == i2, posm, 0.0), axis=1, keepdims=True)
    roff1 = jnp.sum(jnp.where(lane == i1, row_off, 0.0), axis=1, keepdims=True)
    roff2 = jnp.sum(jnp.where(lane == i2, row_off, 0.0), axis=1, keepdims=True)
    keep1 = pos1 < CAP
    keep2 = pos2 < CAP
    dst1 = jnp.where(keep1, roff1 + pos1, float(TRASH)).astype(jnp.int32)
    dst2 = jnp.where(keep2, roff2 + pos2, float(TRASH)).astype(jnp.int32)

    col2 = lax.broadcasted_iota(jnp.int32, (T, K), 1)
    dst_ref[...] = jnp.where(col2 == 0, dst1, dst2)

    g1 = jnp.where(keep1, m1, 0.0)
    g2 = jnp.where(keep2, m2, 0.0)
    col32 = lax.broadcasted_iota(jnp.int32, (T, 2 * E), 1)
    gw_ref[...] = jnp.where(col32 < E, g1, g2)

    bi = lax.broadcasted_iota(jnp.int32, (NB, E), 0).astype(jnp.float32)
    in_blk = (bi >= blk_start) & (bi < blk_start + nb_e)
    lane_r = lax.broadcasted_iota(jnp.int32, (NB, E), 1)
    be_raw = jnp.sum(jnp.where(in_blk, lane_r, 0), axis=1, keepdims=True)
    last_e = jnp.max(jnp.where(nb_e > 0, lane16, -1))
    bcol = lax.broadcasted_iota(jnp.int32, (NB, 1), 0).astype(jnp.float32)
    be = jnp.where(bcol < nbl, be_raw, last_e)
    nbl_col = jnp.full((NB, 1), nbl).astype(jnp.int32)
    meta_ref[...] = jnp.concatenate([be, nbl_col], axis=0)


_router = pl.pallas_call(
    _router_body,
    out_shape=(
        jax.ShapeDtypeStruct((T, K), jnp.int32),
        jax.ShapeDtypeStruct((T, 2 * E), jnp.float32),
        jax.ShapeDtypeStruct((2 * NB, 1), jnp.int32),
    ),
)


def _dispatch_body(x_hbm, dst_hbm, xin_hbm, idx_v, tok_v, rows_v, sem):
    wid = lax.axis_index("s") * NUM_SC_CORES + lax.axis_index("c")
    base = wid * APW
    for c in range(APW // ACH):
        a0 = base + c * ACH
        pltpu.sync_copy(dst_hbm.at[pl.ds(a0, ACH)], idx_v)
        for j in range(ACH // 16):
            toks = (a0 + j * 16 + lax.iota(jnp.int32, 16)) >> 1
            tok_v[pl.ds(j * 16, 16)] = toks
        pltpu.async_copy(x_hbm.at[tok_v], rows_v, sem).wait()
        pltpu.async_copy(rows_v, xin_hbm.at[idx_v], sem).wait()


@functools.lru_cache(maxsize=None)
def _get_dispatch():
    return pl.kernel(
        _dispatch_body,
        out_type=jax.ShapeDtypeStruct((RT, H), jnp.float32),
        mesh=plsc.VectorSubcoreMesh(
            core_axis_name="c", subcore_axis_name="s",
            num_cores=NUM_SC_CORES, num_subcores=NUM_SC_SUBCORES),
        scratch_types=[
            pltpu.VMEM((ACH,), jnp.int32),
            pltpu.VMEM((ACH,), jnp.int32),
            pltpu.VMEM((ACH, H), jnp.float32),
            pltpu.SemaphoreType.DMA,
        ],
    )


def _ffn_body(meta_ref, xin_ref, w1_ref, w2_ref, out_ref):
    b = pl.program_id(0)
    f = pl.program_id(1)
    nbl = meta_ref[NB]

    @pl.when(b < nbl)
    def _():
        h = jnp.dot(xin_ref[...], w1_ref[...], preferred_element_type=jnp.float32)
        h = 0.5 * h * (1.0 + lax.erf(h * (2.0 ** -0.5)))
        acc = jnp.dot(h, w2_ref[...], preferred_element_type=jnp.float32)

        @pl.when(f == 0)
        def _():
            out_ref[...] = acc

        @pl.when(f > 0)
        def _():
            out_ref[...] += acc


def _ffn_xin_map(b, f, m):
    return (jnp.minimum(b, m[NB] - 1), 0)


def _ffn_w1_map(b, f, m):
    return (0, m[b] * NF + jnp.where(b < m[NB], f, NF - 1))


def _ffn_w2_map(b, f, m):
    return (m[b] * NF + jnp.where(b < m[NB], f, NF - 1), 0)


_ffn = pl.pallas_call(
    _ffn_body,
    grid_spec=pltpu.PrefetchScalarGridSpec(
        num_scalar_prefetch=1,
        grid=(NB, NF),
        in_specs=[
            pl.BlockSpec((BC, H), _ffn_xin_map),
            pl.BlockSpec((H, BF), _ffn_w1_map),
            pl.BlockSpec((BF, H), _ffn_w2_map),
        ],
        out_specs=pl.BlockSpec((BC, H), _ffn_xin_map),
    ),
    out_shape=jax.ShapeDtypeStruct((NB * BC, H), jnp.float32),
)


def _combine_body(yout_hbm, dst_hbm, gw_hbm, out_hbm,
                  srci_v, gw_v, rows_v, o_v, sem):
    wid = lax.axis_index("s") * NUM_SC_CORES + lax.axis_index("c")
    base_a = wid * APW
    base_t = wid * (APW // K)
    for c in range(APW // ACH):
        a0 = base_a + c * ACH
        t0 = base_t + c * TCH
        pltpu.sync_copy(dst_hbm.at[pl.ds(a0, ACH)], srci_v)
        for j in range(ACH // 16):
            v = srci_v[pl.ds(j * 16, 16)]
            srci_v[pl.ds(j * 16, 16)] = jnp.where(v >= TRASH, 0, v)
        pltpu.sync_copy(gw_hbm.at[pl.ds(t0, TCH)], gw_v)
        pltpu.async_copy(yout_hbm.at[srci_v], rows_v, sem).wait()

        def tok_body(k, _):
            g0 = gw_v[k, pl.ds(0, 16)]
            g1 = gw_v[k, pl.ds(16, 16)]
            for cc in range(H // 16):
                r0 = rows_v[2 * k, pl.ds(cc * 16, 16)]
                r1 = rows_v[2 * k + 1, pl.ds(cc * 16, 16)]
                o_v[k, pl.ds(cc * 16, 16)] = g0 * r0 + g1 * r1
            return 0

        lax.fori_loop(0, TCH, tok_body, 0)
        pltpu.sync_copy(o_v, out_hbm.at[pl.ds(t0, TCH)])


@functools.lru_cache(maxsize=None)
def _get_combine():
    return pl.kernel(
        _combine_body,
        out_type=jax.ShapeDtypeStruct((T, H), jnp.float32),
        mesh=plsc.VectorSubcoreMesh(
            core_axis_name="c", subcore_axis_name="s",
            num_cores=NUM_SC_CORES, num_subcores=NUM_SC_SUBCORES),
        scratch_types=[
            pltpu.VMEM((ACH,), jnp.int32),
            pltpu.VMEM((TCH, 2 * E), jnp.float32),
            pltpu.VMEM((ACH, H), jnp.float32),
            pltpu.VMEM((TCH, H), jnp.float32),
            pltpu.SemaphoreType.DMA,
        ],
    )


def kernel(x, router_weight, w1, w2):
    dst2, gw, meta = _router(x, router_weight)
    dst = dst2.reshape(T * K)
    meta_flat = meta.reshape(2 * NB)
    xin = _get_dispatch()(x, dst)
    yout = _ffn(meta_flat, xin, w1, w2)
    out = _get_combine()(yout, dst, gw)
    return out


# speedup vs baseline: 2.0869x; 1.1306x over previous
import functools

import jax
import jax.numpy as jnp
from jax import lax
from jax.experimental import pallas as pl
from jax.experimental.pallas import tpu as pltpu
from jax.experimental.pallas import tpu_sc as plsc

T = 2048
H = 1024
F = 2048
E = 16
K = 2
CAP = (T * K // E) * 2

BC = 256
BF = 1024
NF = F // BF
NB = 32
RT = (NB + 1) * BC
TRASH = NB * BC

NUM_SC_CORES = 2
NUM_SC_SUBCORES = 16
NW = NUM_SC_CORES * NUM_SC_SUBCORES
APW = T * K // NW
ACH = 32
TCH = ACH // K


def _router_body(x_ref, rw_ref, dst_ref, gw_ref, meta_ref):
    xf = x_ref[...]
    logits = jnp.dot(xf, rw_ref[...], preferred_element_type=jnp.float32)
    m = jnp.max(logits, axis=1, keepdims=True)
    p = jnp.exp(logits - m)
    p = p / jnp.sum(p, axis=1, keepdims=True)

    lane = lax.broadcasted_iota(jnp.int32, (T, E), 1)
    m1 = jnp.max(p, axis=1, keepdims=True)
    i1 = jnp.min(jnp.where(p == m1, lane, E), axis=1, keepdims=True)
    p2 = jnp.where(lane == i1, -1.0, p)
    m2 = jnp.max(p2, axis=1, keepdims=True)
    i2 = jnp.min(jnp.where(p2 == m2, lane, E), axis=1, keepdims=True)

    oh = (lane == i1).astype(jnp.float32) + (lane == i2).astype(jnp.float32)
    base = jnp.zeros((1, E), jnp.float32)
    chunks = []
    CH = 512
    ri = lax.broadcasted_iota(jnp.int32, (CH, CH), 0)
    ci = lax.broadcasted_iota(jnp.int32, (CH, CH), 1)
    tril = (ci < ri).astype(jnp.float32)
    for c in range(T // CH):
        ohc = oh[c * CH:(c + 1) * CH, :]
        excl = jnp.dot(tril, ohc, preferred_element_type=jnp.float32) + base
        chunks.append(excl)
        base = base + jnp.sum(ohc, axis=0, keepdims=True)
    posm = jnp.concatenate(chunks, axis=0)
    counts = base
    countc = jnp.minimum(counts, float(CAP))

    nb_e = jnp.ceil(countc / BC)
    lane16 = lax.broadcasted_iota(jnp.int32, (1, E), 1)
    ue = lax.broadcasted_iota(jnp.int32, (E, E), 0)
    ve = lax.broadcasted_iota(jnp.int32, (E, E), 1)
    umat = (ue < ve).astype(jnp.float32)
    blk_start = jnp.dot(nb_e, umat, preferred_element_type=jnp.float32)
    nbl = jnp.sum(nb_e)
    row_off = BC * blk_start

    pos1 = jnp.sum(jnp.where(lane == i1, posm, 0.0), axis=1, keepdims=True)
    pos2 = jnp.sum(jnp.where(lane == i2, posm, 0.0), axis=1, keepdims=True)
    roff1 = jnp.sum(jnp.where(lane == i1, row_off, 0.0), axis=1, keepdims=True)
    roff2 = jnp.sum(jnp.where(lane == i2, row_off, 0.0), axis=1, keepdims=True)
    keep1 = pos1 < CAP
    keep2 = pos2 < CAP
    dst1 = jnp.where(keep1, roff1 + pos1, float(TRASH)).astype(jnp.int32)
    dst2 = jnp.where(keep2, roff2 + pos2, float(TRASH)).astype(jnp.int32)

    col2 = lax.broadcasted_iota(jnp.int32, (T, K), 1)
    dst_ref[...] = jnp.where(col2 == 0, dst1, dst2)

    g1 = jnp.where(keep1, m1, 0.0)
    g2 = jnp.where(keep2, m2, 0.0)
    col32 = lax.broadcasted_iota(jnp.int32, (T, 2 * E), 1)
    gw_ref[...] = jnp.where(col32 < E, g1, g2)

    bi = lax.broadcasted_iota(jnp.int32, (NB, E), 0).astype(jnp.float32)
    in_blk = (bi >= blk_start) & (bi < blk_start + nb_e)
    lane_r = lax.broadcasted_iota(jnp.int32, (NB, E), 1)
    be_raw = jnp.sum(jnp.where(in_blk, lane_r, 0), axis=1, keepdims=True)
    last_e = jnp.max(jnp.where(nb_e > 0, lane16, -1))
    bcol = lax.broadcasted_iota(jnp.int32, (NB, 1), 0).astype(jnp.float32)
    be = jnp.where(bcol < nbl, be_raw, last_e)
    nbl_col = jnp.full((NB, 1), nbl).astype(jnp.int32)
    meta_ref[...] = jnp.concatenate([be, nbl_col], axis=0)


_router = pl.pallas_call(
    _router_body,
    out_shape=(
        jax.ShapeDtypeStruct((T, K), jnp.int32),
        jax.ShapeDtypeStruct((T, 2 * E), jnp.float32),
        jax.ShapeDtypeStruct((2 * NB, 1), jnp.int32),
    ),
)


def _dispatch_body(x_hbm, dst_hbm, xin_hbm, idx_v, tok_v, rows_v, sem):
    wid = lax.axis_index("s") * NUM_SC_CORES + lax.axis_index("c")
    base = wid * APW
    for c in range(APW // ACH):
        a0 = base + c * ACH
        pltpu.sync_copy(dst_hbm.at[pl.ds(a0, ACH)], idx_v)
        for j in range(ACH // 16):
            toks = (a0 + j * 16 + lax.iota(jnp.int32, 16)) >> 1
            tok_v[pl.ds(j * 16, 16)] = toks
        pltpu.async_copy(x_hbm.at[tok_v], rows_v, sem).wait()
        pltpu.async_copy(rows_v, xin_hbm.at[idx_v], sem).wait()


@functools.lru_cache(maxsize=None)
def _get_dispatch():
    return pl.kernel(
        _dispatch_body,
        out_type=jax.ShapeDtypeStruct((RT, H), jnp.float32),
        mesh=plsc.VectorSubcoreMesh(
            core_axis_name="c", subcore_axis_name="s",
            num_cores=NUM_SC_CORES, num_subcores=NUM_SC_SUBCORES),
        scratch_types=[
            pltpu.VMEM((ACH,), jnp.int32),
            pltpu.VMEM((ACH,), jnp.int32),
            pltpu.VMEM((ACH, H), jnp.float32),
            pltpu.SemaphoreType.DMA,
        ],
    )


def _ffn_body(meta_ref, xin_ref, w1_ref, w2_ref, out_ref):
    b = pl.program_id(0)
    f = pl.program_id(1)
    nbl = meta_ref[NB]

    @pl.when(b < nbl)
    def _():
        h = jnp.dot(xin_ref[...].astype(jnp.bfloat16),
                    w1_ref[...].astype(jnp.bfloat16),
                    preferred_element_type=jnp.float32)
        h = 0.5 * h * (1.0 + lax.erf(h * (2.0 ** -0.5)))
        acc = jnp.dot(h.astype(jnp.bfloat16),
                      w2_ref[...].astype(jnp.bfloat16),
                      preferred_element_type=jnp.float32)

        @pl.when(f == 0)
        def _():
            out_ref[...] = acc

        @pl.when(f > 0)
        def _():
            out_ref[...] += acc


def _ffn_xin_map(b, f, m):
    return (jnp.minimum(b, m[NB] - 1), 0)


def _ffn_w1_map(b, f, m):
    return (0, m[b] * NF + jnp.where(b < m[NB], f, NF - 1))


def _ffn_w2_map(b, f, m):
    return (m[b] * NF + jnp.where(b < m[NB], f, NF - 1), 0)


_ffn = pl.pallas_call(
    _ffn_body,
    grid_spec=pltpu.PrefetchScalarGridSpec(
        num_scalar_prefetch=1,
        grid=(NB, NF),
        in_specs=[
            pl.BlockSpec((BC, H), _ffn_xin_map),
            pl.BlockSpec((H, BF), _ffn_w1_map),
            pl.BlockSpec((BF, H), _ffn_w2_map),
        ],
        out_specs=pl.BlockSpec((BC, H), _ffn_xin_map),
    ),
    out_shape=jax.ShapeDtypeStruct((NB * BC, H), jnp.float32),
)


def _combine_body(yout_hbm, dst_hbm, gw_hbm, out_hbm,
                  srci_v, gw_v, rows_v, o_v, sem):
    wid = lax.axis_index("s") * NUM_SC_CORES + lax.axis_index("c")
    base_a = wid * APW
    base_t = wid * (APW // K)
    for c in range(APW // ACH):
        a0 = base_a + c * ACH
        t0 = base_t + c * TCH
        pltpu.sync_copy(dst_hbm.at[pl.ds(a0, ACH)], srci_v)
        for j in range(ACH // 16):
            v = srci_v[pl.ds(j * 16, 16)]
            srci_v[pl.ds(j * 16, 16)] = jnp.where(v >= TRASH, 0, v)
        pltpu.sync_copy(gw_hbm.at[pl.ds(t0, TCH)], gw_v)
        pltpu.async_copy(yout_hbm.at[srci_v], rows_v, sem).wait()

        def tok_body(k, _):
            g0 = gw_v[k, pl.ds(0, 16)]
            g1 = gw_v[k, pl.ds(16, 16)]
            for cc in range(H // 16):
                r0 = rows_v[2 * k, pl.ds(cc * 16, 16)]
                r1 = rows_v[2 * k + 1, pl.ds(cc * 16, 16)]
                o_v[k, pl.ds(cc * 16, 16)] = g0 * r0 + g1 * r1
            return 0

        lax.fori_loop(0, TCH, tok_body, 0)
        pltpu.sync_copy(o_v, out_hbm.at[pl.ds(t0, TCH)])


@functools.lru_cache(maxsize=None)
def _get_combine():
    return pl.kernel(
        _combine_body,
        out_type=jax.ShapeDtypeStruct((T, H), jnp.float32),
        mesh=plsc.VectorSubcoreMesh(
            core_axis_name="c", subcore_axis_name="s",
            num_cores=NUM_SC_CORES, num_subcores=NUM_SC_SUBCORES),
        scratch_types=[
            pltpu.VMEM((ACH,), jnp.int32),
            pltpu.VMEM((TCH, 2 * E), jnp.float32),
            pltpu.VMEM((ACH, H), jnp.float32),
            pltpu.VMEM((TCH, H), jnp.float32),
            pltpu.SemaphoreType.DMA,
        ],
    )


def kernel(x, router_weight, w1, w2):
    dst2, gw, meta = _router(x, router_weight)
    dst = dst2.reshape(T * K)
    meta_flat = meta.reshape(2 * NB)
    xin = _get_dispatch()(x, dst)
    yout = _ffn(meta_flat, xin, w1, w2)
    out = _get_combine()(yout, dst, gw)
    return out


# speedup vs baseline: 2.1556x; 1.0329x over previous
import functools

import jax
import jax.numpy as jnp
from jax import lax
from jax.experimental import pallas as pl
from jax.experimental.pallas import tpu as pltpu
from jax.experimental.pallas import tpu_sc as plsc

T = 2048
H = 1024
F = 2048
E = 16
K = 2
CAP = (T * K // E) * 2

BC = 256
BF = 1024
NF = F // BF
NB = 32
NBP = NB + 1
RT = NBP * BC
TRASH = NB * BC
GW = 128

NUM_SC_CORES = 2
NUM_SC_SUBCORES = 16
NW = NUM_SC_CORES * NUM_SC_SUBCORES
APW = T * K // NW
ACH = 32
NCH = APW // ACH
TCH = ACH // K


def _router_body(x_ref, rw_ref, dst_ref, gw_ref, meta_ref):
    xf = x_ref[...]
    logits = jnp.dot(xf, rw_ref[...], preferred_element_type=jnp.float32)
    m = jnp.max(logits, axis=1, keepdims=True)
    p = jnp.exp(logits - m)
    p = p / jnp.sum(p, axis=1, keepdims=True)

    lane = lax.broadcasted_iota(jnp.int32, (T, E), 1)
    m1 = jnp.max(p, axis=1, keepdims=True)
    i1 = jnp.min(jnp.where(p == m1, lane, E), axis=1, keepdims=True)
    p2 = jnp.where(lane == i1, -1.0, p)
    m2 = jnp.max(p2, axis=1, keepdims=True)
    i2 = jnp.min(jnp.where(p2 == m2, lane, E), axis=1, keepdims=True)

    oh = (lane == i1).astype(jnp.float32) + (lane == i2).astype(jnp.float32)
    base = jnp.zeros((1, E), jnp.float32)
    chunks = []
    CH = 512
    ri = lax.broadcasted_iota(jnp.int32, (CH, CH), 0)
    ci = lax.broadcasted_iota(jnp.int32, (CH, CH), 1)
    tril = (ci < ri).astype(jnp.float32)
    for c in range(T // CH):
        ohc = oh[c * CH:(c + 1) * CH, :]
        excl = jnp.dot(tril, ohc, preferred_element_type=jnp.float32) + base
        chunks.append(excl)
        base = base + jnp.sum(ohc, axis=0, keepdims=True)
    posm = jnp.concatenate(chunks, axis=0)
    counts = base
    countc = jnp.minimum(counts, float(CAP))

    nb_e = jnp.ceil(countc / BC)
    lane16 = lax.broadcasted_iota(jnp.int32, (1, E), 1)
    ue = lax.broadcasted_iota(jnp.int32, (E, E), 0)
    ve = lax.broadcasted_iota(jnp.int32, (E, E), 1)
    umat = (ue < ve).astype(jnp.float32)
    blk_start = jnp.dot(nb_e, umat, preferred_element_type=jnp.float32)
    nbl = jnp.sum(nb_e)
    row_off = BC * blk_start

    pos1 = jnp.sum(jnp.where(lane == i1, posm, 0.0), axis=1, keepdims=True)
    pos2 = jnp.sum(jnp.where(lane == i2, posm, 0.0), axis=1, keepdims=True)
    roff1 = jnp.sum(jnp.where(lane == i1, row_off, 0.0), axis=1, keepdims=True)
    roff2 = jnp.sum(jnp.where(lane == i2, row_off, 0.0), axis=1, keepdims=True)
    keep1 = pos1 < CAP
    keep2 = pos2 < CAP
    dst1 = jnp.where(keep1, roff1 + pos1, float(TRASH)).astype(jnp.int32)
    dst2 = jnp.where(keep2, roff2 + pos2, float(TRASH)).astype(jnp.int32)

    col2 = lax.broadcasted_iota(jnp.int32, (T, K), 1)
    dst_ref[...] = jnp.where(col2 == 0, dst1, dst2)

    g1 = jnp.where(keep1, m1, 0.0)
    g2 = jnp.where(keep2, m2, 0.0)
    colg = lax.broadcasted_iota(jnp.int32, (T, 2 * GW), 1)
    gw_ref[...] = jnp.where(colg < GW, g1, g2)

    bi = lax.broadcasted_iota(jnp.int32, (NBP, E), 0).astype(jnp.float32)
    in_blk = (bi >= blk_start) & (bi < blk_start + nb_e)
    lane_r = lax.broadcasted_iota(jnp.int32, (NBP, E), 1)
    be_raw = jnp.sum(jnp.where(in_blk, lane_r, 0), axis=1, keepdims=True)
    last_e = jnp.max(jnp.where(nb_e > 0, lane16, -1))
    bcol = lax.broadcasted_iota(jnp.int32, (NBP, 1), 0).astype(jnp.float32)
    be = jnp.where(bcol < nbl, be_raw, last_e)
    nbl_col = jnp.full((NBP, 1), nbl).astype(jnp.int32)
    meta_ref[...] = jnp.concatenate([be, nbl_col], axis=0)


_router = pl.pallas_call(
    _router_body,
    out_shape=(
        jax.ShapeDtypeStruct((T, K), jnp.int32),
        jax.ShapeDtypeStruct((T, 2 * GW), jnp.float32),
        jax.ShapeDtypeStruct((2 * NBP, 1), jnp.int32),
    ),
)


def _dispatch_body(x_hbm, dst_hbm, gw2_hbm, xin_hbm, gatec_hbm,
                   idxall_v, gwa_v, idx0, idx1, tok0, tok1, rows0, rows1,
                   gsem0, gsem1, ssem0, ssem1):
    wid = lax.axis_index("s") * NUM_SC_CORES + lax.axis_index("c")
    base = wid * APW

    pltpu.sync_copy(dst_hbm.at[pl.ds(base, APW)], idxall_v)
    pltpu.sync_copy(gw2_hbm.at[pl.ds(base, APW)], gwa_v)
    pltpu.sync_copy(gwa_v, gatec_hbm.at[idxall_v])

    idxb = (idx0, idx1)
    tokb = (tok0, tok1)
    rowsb = (rows0, rows1)
    gsem = (gsem0, gsem1)
    ssem = (ssem0, ssem1)
    gd = [None] * NCH
    sd = [None] * NCH

    def prep(c):
        b = c % 2
        a0 = base + c * ACH
        pltpu.sync_copy(dst_hbm.at[pl.ds(a0, ACH)], idxb[b])
        for j in range(ACH // 16):
            tokb[b][pl.ds(j * 16, 16)] = (a0 + j * 16 + lax.iota(jnp.int32, 16)) >> 1
        gd[c] = pltpu.async_copy(x_hbm.at[tokb[b]], rowsb[b], gsem[b])

    prep(0)
    prep(1)
    for c in range(NCH):
        b = c % 2
        gd[c].wait()
        sd[c] = pltpu.async_copy(rowsb[b], xin_hbm.at[idxb[b]], ssem[b])
        if 1 <= c < NCH - 1:
            sd[c - 1].wait()
            prep(c + 1)
    for c in range(max(0, NCH - 2), NCH):
        sd[c].wait()


@functools.lru_cache(maxsize=None)
def _get_dispatch():
    return pl.kernel(
        _dispatch_body,
        out_type=(
            jax.ShapeDtypeStruct((RT, H), jnp.float32),
            jax.ShapeDtypeStruct((RT, GW), jnp.float32),
        ),
        mesh=plsc.VectorSubcoreMesh(
            core_axis_name="c", subcore_axis_name="s",
            num_cores=NUM_SC_CORES, num_subcores=NUM_SC_SUBCORES),
        scratch_types=[
            pltpu.VMEM((APW,), jnp.int32),
            pltpu.VMEM((APW, GW), jnp.float32),
            pltpu.VMEM((ACH,), jnp.int32),
            pltpu.VMEM((ACH,), jnp.int32),
            pltpu.VMEM((ACH,), jnp.int32),
            pltpu.VMEM((ACH,), jnp.int32),
            pltpu.VMEM((ACH, H), jnp.float32),
            pltpu.VMEM((ACH, H), jnp.float32),
            pltpu.SemaphoreType.DMA,
            pltpu.SemaphoreType.DMA,
            pltpu.SemaphoreType.DMA,
            pltpu.SemaphoreType.DMA,
        ],
    )


def _ffn_body(meta_ref, xin_ref, w1_ref, w2_ref, gc_ref, out_ref):
    b = pl.program_id(0)
    f = pl.program_id(1)
    nbl = meta_ref[NBP]

    @pl.when(b < nbl)
    def _():
        h = jnp.dot(xin_ref[...].astype(jnp.bfloat16),
                    w1_ref[...].astype(jnp.bfloat16),
                    preferred_element_type=jnp.float32)
        h = 0.5 * h * (1.0 + lax.erf(h * (2.0 ** -0.5)))
        acc = jnp.dot(h.astype(jnp.bfloat16),
                      w2_ref[...].astype(jnp.bfloat16),
                      preferred_element_type=jnp.float32)

        @pl.when(f == 0)
        def _():
            out_ref[...] = acc

        @pl.when(f == NF - 1)
        def _():
            g = gc_ref[...][:, 0:1]
            prev = acc if NF == 1 else out_ref[...] + acc
            out_ref[...] = prev * g

        if NF > 2:
            @pl.when((f > 0) & (f < NF - 1))
            def _():
                out_ref[...] += acc

    @pl.when(b == NB)
    def _():
        @pl.when(f == 0)
        def _():
            out_ref[...] = jnp.zeros_like(out_ref)


def _act_map(b, f, m):
    return jnp.minimum(b, m[NBP] - 1)


def _ffn_xin_map(b, f, m):
    return (_act_map(b, f, m), 0)


def _ffn_w1_map(b, f, m):
    return (0, m[b] * NF + jnp.where(b < m[NBP], f, NF - 1))


def _ffn_w2_map(b, f, m):
    return (m[b] * NF + jnp.where(b < m[NBP], f, NF - 1), 0)


def _ffn_gc_map(b, f, m):
    return (_act_map(b, f, m), 0)


def _ffn_out_map(b, f, m):
    return (jnp.where(b == NB, NB, _act_map(b, f, m)), 0)


_ffn = pl.pallas_call(
    _ffn_body,
    grid_spec=pltpu.PrefetchScalarGridSpec(
        num_scalar_prefetch=1,
        grid=(NBP, NF),
        in_specs=[
            pl.BlockSpec((BC, H), _ffn_xin_map),
            pl.BlockSpec((H, BF), _ffn_w1_map),
            pl.BlockSpec((BF, H), _ffn_w2_map),
            pl.BlockSpec((BC, GW), _ffn_gc_map),
        ],
        out_specs=pl.BlockSpec((BC, H), _ffn_out_map),
    ),
    out_shape=jax.ShapeDtypeStruct((RT, H), jnp.float32),
)


def _combine_body(yout_hbm, dst_hbm, out_hbm,
                  idx0, idx1, rows0, rows1, o_v, gsem0, gsem1):
    wid = lax.axis_index("s") * NUM_SC_CORES + lax.axis_index("c")
    base_a = wid * APW
    base_t = wid * (APW // K)

    idxb = (idx0, idx1)
    rowsb = (rows0, rows1)
    gsem = (gsem0, gsem1)
    gd = [None] * NCH

    def prep(c):
        b = c % 2
        a0 = base_a + c * ACH
        pltpu.sync_copy(dst_hbm.at[pl.ds(a0, ACH)], idxb[b])
        gd[c] = pltpu.async_copy(yout_hbm.at[idxb[b]], rowsb[b], gsem[b])

    prep(0)
    prep(1)
    for c in range(NCH):
        b = c % 2
        t0 = base_t + c * TCH
        gd[c].wait()
        rows = rowsb[b]

        def tok_body(k, _):
            for cc in range(H // 16):
                r0 = rows[2 * k, pl.ds(cc * 16, 16)]
                r1 = rows[2 * k + 1, pl.ds(cc * 16, 16)]
                o_v[k, pl.ds(cc * 16, 16)] = r0 + r1
            return 0

        lax.fori_loop(0, TCH, tok_body, 0)
        pltpu.sync_copy(o_v, out_hbm.at[pl.ds(t0, TCH)])
        if c + 2 < NCH:
            prep(c + 2)


@functools.lru_cache(maxsize=None)
def _get_combine():
    return pl.kernel(
        _combine_body,
        out_type=jax.ShapeDtypeStruct((T, H), jnp.float32),
        mesh=plsc.VectorSubcoreMesh(
            core_axis_name="c", subcore_axis_name="s",
            num_cores=NUM_SC_CORES, num_subcores=NUM_SC_SUBCORES),
        scratch_types=[
            pltpu.VMEM((ACH,), jnp.int32),
            pltpu.VMEM((ACH,), jnp.int32),
            pltpu.VMEM((ACH, H), jnp.float32),
            pltpu.VMEM((ACH, H), jnp.float32),
            pltpu.VMEM((TCH, H), jnp.float32),
            pltpu.SemaphoreType.DMA,
            pltpu.SemaphoreType.DMA,
        ],
    )


def kernel(x, router_weight, w1, w2):
    dst2, gw, meta = _router(x, router_weight)
    dst = dst2.reshape(T * K)
    gw2 = gw.reshape(T * K, GW)
    meta_flat = meta.reshape(2 * NBP)
    xin, gatec = _get_dispatch()(x, dst, gw2)
    yout = _ffn(meta_flat, xin, w1, w2, gatec)
    out = _get_combine()(yout, dst)
    return out


# speedup vs baseline: 2.3470x; 1.0888x over previous
import functools

import jax
import jax.numpy as jnp
from jax import lax
from jax.experimental import pallas as pl
from jax.experimental.pallas import tpu as pltpu
from jax.experimental.pallas import tpu_sc as plsc

T = 2048
H = 1024
F = 2048
E = 16
K = 2
CAP = (T * K // E) * 2

BC = 256
BF = 2048
NF = F // BF
NB = 32
NBP = NB + 1
RT = NBP * BC
TRASH = NB * BC
GW = 128

NUM_SC_CORES = 2
NUM_SC_SUBCORES = 16
NW = NUM_SC_CORES * NUM_SC_SUBCORES
APW = T * K // NW
ACH = 32
NCH = APW // ACH
TCH = ACH // K


def _router_body(x_ref, rw_ref, dst_ref, gw_ref, meta_ref):
    xf = x_ref[...]
    logits = jnp.dot(xf, rw_ref[...], preferred_element_type=jnp.float32)
    m = jnp.max(logits, axis=1, keepdims=True)
    p = jnp.exp(logits - m)
    p = p / jnp.sum(p, axis=1, keepdims=True)

    lane = lax.broadcasted_iota(jnp.int32, (T, E), 1)
    m1 = jnp.max(p, axis=1, keepdims=True)
    i1 = jnp.min(jnp.where(p == m1, lane, E), axis=1, keepdims=True)
    p2 = jnp.where(lane == i1, -1.0, p)
    m2 = jnp.max(p2, axis=1, keepdims=True)
    i2 = jnp.min(jnp.where(p2 == m2, lane, E), axis=1, keepdims=True)

    oh = (lane == i1).astype(jnp.float32) + (lane == i2).astype(jnp.float32)
    base = jnp.zeros((1, E), jnp.float32)
    chunks = []
    CH = 512
    ri = lax.broadcasted_iota(jnp.int32, (CH, CH), 0)
    ci = lax.broadcasted_iota(jnp.int32, (CH, CH), 1)
    tril = (ci < ri).astype(jnp.float32)
    for c in range(T // CH):
        ohc = oh[c * CH:(c + 1) * CH, :]
        excl = jnp.dot(tril, ohc, preferred_element_type=jnp.float32) + base
        chunks.append(excl)
        base = base + jnp.sum(ohc, axis=0, keepdims=True)
    posm = jnp.concatenate(chunks, axis=0)
    counts = base
    countc = jnp.minimum(counts, float(CAP))

    nb_e = jnp.ceil(countc / BC)
    lane16 = lax.broadcasted_iota(jnp.int32, (1, E), 1)
    ue = lax.broadcasted_iota(jnp.int32, (E, E), 0)
    ve = lax.broadcasted_iota(jnp.int32, (E, E), 1)
    umat = (ue < ve).astype(jnp.float32)
    blk_start = jnp.dot(nb_e, umat, preferred_element_type=jnp.float32)
    nbl = jnp.sum(nb_e)
    row_off = BC * blk_start

    pos1 = jnp.sum(jnp.where(lane == i1, posm, 0.0), axis=1, keepdims=True)
    pos2 = jnp.sum(jnp.where(lane == i2, posm, 0.0), axis=1, keepdims=True)
    roff1 = jnp.sum(jnp.where(lane == i1, row_off, 0.0), axis=1, keepdims=True)
    roff2 = jnp.sum(jnp.where(lane == i2, row_off, 0.0), axis=1, keepdims=True)
    keep1 = pos1 < CAP
    keep2 = pos2 < CAP
    dst1 = jnp.where(keep1, roff1 + pos1, float(TRASH)).astype(jnp.int32)
    dst2 = jnp.where(keep2, roff2 + pos2, float(TRASH)).astype(jnp.int32)

    col2 = lax.broadcasted_iota(jnp.int32, (T, K), 1)
    dst_ref[...] = jnp.where(col2 == 0, dst1, dst2)

    g1 = jnp.where(keep1, m1, 0.0)
    g2 = jnp.where(keep2, m2, 0.0)
    colg = lax.broadcasted_iota(jnp.int32, (T, 2 * GW), 1)
    gw_ref[...] = jnp.where(colg < GW, g1, g2)

    bi = lax.broadcasted_iota(jnp.int32, (NBP, E), 0).astype(jnp.float32)
    in_blk = (bi >= blk_start) & (bi < blk_start + nb_e)
    lane_r = lax.broadcasted_iota(jnp.int32, (NBP, E), 1)
    be_raw = jnp.sum(jnp.where(in_blk, lane_r, 0), axis=1, keepdims=True)
    last_e = jnp.max(jnp.where(nb_e > 0, lane16, -1))
    bcol = lax.broadcasted_iota(jnp.int32, (NBP, 1), 0).astype(jnp.float32)
    be = jnp.where(bcol < nbl, be_raw, last_e)
    nbl_col = jnp.full((NBP, 1), nbl).astype(jnp.int32)
    meta_ref[...] = jnp.concatenate([be, nbl_col], axis=0)


_router = pl.pallas_call(
    _router_body,
    out_shape=(
        jax.ShapeDtypeStruct((T, K), jnp.int32),
        jax.ShapeDtypeStruct((T, 2 * GW), jnp.float32),
        jax.ShapeDtypeStruct((2 * NBP, 1), jnp.int32),
    ),
)


def _dispatch_body(x_hbm, dst_hbm, gw2_hbm, xin_hbm, gatec_hbm,
                   idxall_v, gwa_v, idx0, idx1, tok0, tok1, rows0, rows1,
                   gsem0, gsem1, ssem0, ssem1):
    wid = lax.axis_index("s") * NUM_SC_CORES + lax.axis_index("c")
    base = wid * APW

    pltpu.sync_copy(dst_hbm.at[pl.ds(base, APW)], idxall_v)
    pltpu.sync_copy(gw2_hbm.at[pl.ds(base, APW)], gwa_v)
    pltpu.sync_copy(gwa_v, gatec_hbm.at[idxall_v])

    idxb = (idx0, idx1)
    tokb = (tok0, tok1)
    rowsb = (rows0, rows1)
    gsem = (gsem0, gsem1)
    ssem = (ssem0, ssem1)
    gd = [None] * NCH
    sd = [None] * NCH

    def prep(c):
        b = c % 2
        a0 = base + c * ACH
        pltpu.sync_copy(dst_hbm.at[pl.ds(a0, ACH)], idxb[b])
        for j in range(ACH // 16):
            tokb[b][pl.ds(j * 16, 16)] = (a0 + j * 16 + lax.iota(jnp.int32, 16)) >> 1
        gd[c] = pltpu.async_copy(x_hbm.at[tokb[b]], rowsb[b], gsem[b])

    prep(0)
    prep(1)
    for c in range(NCH):
        b = c % 2
        gd[c].wait()
        sd[c] = pltpu.async_copy(rowsb[b], xin_hbm.at[idxb[b]], ssem[b])
        if 1 <= c < NCH - 1:
            sd[c - 1].wait()
            prep(c + 1)
    for c in range(max(0, NCH - 2), NCH):
        sd[c].wait()


@functools.lru_cache(maxsize=None)
def _get_dispatch():
    return pl.kernel(
        _dispatch_body,
        out_type=(
            jax.ShapeDtypeStruct((RT, H), jnp.float32),
            jax.ShapeDtypeStruct((RT, GW), jnp.float32),
        ),
        mesh=plsc.VectorSubcoreMesh(
            core_axis_name="c", subcore_axis_name="s",
            num_cores=NUM_SC_CORES, num_subcores=NUM_SC_SUBCORES),
        scratch_types=[
            pltpu.VMEM((APW,), jnp.int32),
            pltpu.VMEM((APW, GW), jnp.float32),
            pltpu.VMEM((ACH,), jnp.int32),
            pltpu.VMEM((ACH,), jnp.int32),
            pltpu.VMEM((ACH,), jnp.int32),
            pltpu.VMEM((ACH,), jnp.int32),
            pltpu.VMEM((ACH, H), jnp.float32),
            pltpu.VMEM((ACH, H), jnp.float32),
            pltpu.SemaphoreType.DMA,
            pltpu.SemaphoreType.DMA,
            pltpu.SemaphoreType.DMA,
            pltpu.SemaphoreType.DMA,
        ],
    )


def _ffn_body(meta_ref, xin_ref, w1_ref, w2_ref, gc_ref, out_ref):
    b = pl.program_id(0)
    nbl = meta_ref[NBP]

    @pl.when(b < nbl)
    def _():
        h = jnp.dot(xin_ref[...].astype(jnp.bfloat16),
                    w1_ref[...].astype(jnp.bfloat16),
                    preferred_element_type=jnp.float32)
        h = 0.5 * h * (1.0 + lax.erf(h * (2.0 ** -0.5)))
        acc = jnp.dot(h.astype(jnp.bfloat16),
                      w2_ref[...].astype(jnp.bfloat16),
                      preferred_element_type=jnp.float32)
        out_ref[...] = acc * gc_ref[...][:, 0:1]

    @pl.when(b == NB)
    def _():
        out_ref[...] = jnp.zeros_like(out_ref)


def _act_map(b, m):
    return jnp.minimum(b, m[NBP] - 1)


def _ffn_xin_map(b, m):
    return (_act_map(b, m), 0)


def _ffn_w1_map(b, m):
    return (0, m[b])


def _ffn_w2_map(b, m):
    return (m[b], 0)


def _ffn_gc_map(b, m):
    return (_act_map(b, m), 0)


def _ffn_out_map(b, m):
    return (jnp.where(b == NB, NB, _act_map(b, m)), 0)


_ffn = pl.pallas_call(
    _ffn_body,
    grid_spec=pltpu.PrefetchScalarGridSpec(
        num_scalar_prefetch=1,
        grid=(NBP,),
        in_specs=[
            pl.BlockSpec((BC, H), _ffn_xin_map),
            pl.BlockSpec((H, BF), _ffn_w1_map),
            pl.BlockSpec((BF, H), _ffn_w2_map),
            pl.BlockSpec((BC, GW), _ffn_gc_map),
        ],
        out_specs=pl.BlockSpec((BC, H), _ffn_out_map),
    ),
    out_shape=jax.ShapeDtypeStruct((RT, H), jnp.float32),
)


def _combine_body(yout_hbm, dst_hbm, out_hbm,
                  idx0, idx1, rows0, rows1, o_v, gsem0, gsem1):
    wid = lax.axis_index("s") * NUM_SC_CORES + lax.axis_index("c")
    base_a = wid * APW
    base_t = wid * (APW // K)

    idxb = (idx0, idx1)
    rowsb = (rows0, rows1)
    gsem = (gsem0, gsem1)
    gd = [None] * NCH

    def prep(c):
        b = c % 2
        a0 = base_a + c * ACH
        pltpu.sync_copy(dst_hbm.at[pl.ds(a0, ACH)], idxb[b])
        gd[c] = pltpu.async_copy(yout_hbm.at[idxb[b]], rowsb[b], gsem[b])

    prep(0)
    prep(1)
    for c in range(NCH):
        b = c % 2
        t0 = base_t + c * TCH
        gd[c].wait()
        rows = rowsb[b]

        def tok_body(k, _):
            for cc in range(H // 16):
                r0 = rows[2 * k, pl.ds(cc * 16, 16)]
                r1 = rows[2 * k + 1, pl.ds(cc * 16, 16)]
                o_v[k, pl.ds(cc * 16, 16)] = r0 + r1
            return 0

        lax.fori_loop(0, TCH, tok_body, 0)
        pltpu.sync_copy(o_v, out_hbm.at[pl.ds(t0, TCH)])
        if c + 2 < NCH:
            prep(c + 2)


@functools.lru_cache(maxsize=None)
def _get_combine():
    return pl.kernel(
        _combine_body,
        out_type=jax.ShapeDtypeStruct((T, H), jnp.float32),
        mesh=plsc.VectorSubcoreMesh(
            core_axis_name="c", subcore_axis_name="s",
            num_cores=NUM_SC_CORES, num_subcores=NUM_SC_SUBCORES),
        scratch_types=[
            pltpu.VMEM((ACH,), jnp.int32),
            pltpu.VMEM((ACH,), jnp.int32),
            pltpu.VMEM((ACH, H), jnp.float32),
            pltpu.VMEM((ACH, H), jnp.float32),
            pltpu.VMEM((TCH, H), jnp.float32),
            pltpu.SemaphoreType.DMA,
            pltpu.SemaphoreType.DMA,
        ],
    )


def kernel(x, router_weight, w1, w2):
    dst2, gw, meta = _router(x, router_weight)
    dst = dst2.reshape(T * K)
    gw2 = gw.reshape(T * K, GW)
    meta_flat = meta.reshape(2 * NBP)
    xin, gatec = _get_dispatch()(x, dst, gw2)
    yout = _ffn(meta_flat, xin, w1, w2, gatec)
    out = _get_combine()(yout, dst)
    return out


# speedup vs baseline: 2.3626x; 1.0066x over previous
import functools

import jax
import jax.numpy as jnp
from jax import lax
from jax.experimental import pallas as pl
from jax.experimental.pallas import tpu as pltpu
from jax.experimental.pallas import tpu_sc as plsc

T = 2048
H = 1024
F = 2048
E = 16
K = 2
CAP = (T * K // E) * 2

BC = 256
BF = 2048
NF = F // BF
NB = 32
NBP = NB + 1
RT = NBP * BC
TRASH = NB * BC
GW = 128

NUM_SC_CORES = 2
NUM_SC_SUBCORES = 16
NW = NUM_SC_CORES * NUM_SC_SUBCORES
APW = T * K // NW
ACH = 32
NCH = APW // ACH
TCH = ACH // K


def _router_body(x_ref, rw_ref, dst_ref, gw_ref, meta_ref):
    xf = x_ref[...]
    logits = jnp.dot(xf, rw_ref[...], preferred_element_type=jnp.float32)
    m = jnp.max(logits, axis=1, keepdims=True)
    p = jnp.exp(logits - m)
    p = p / jnp.sum(p, axis=1, keepdims=True)

    lane = lax.broadcasted_iota(jnp.int32, (T, E), 1)
    m1 = jnp.max(p, axis=1, keepdims=True)
    i1 = jnp.min(jnp.where(p == m1, lane, E), axis=1, keepdims=True)
    p2 = jnp.where(lane == i1, -1.0, p)
    m2 = jnp.max(p2, axis=1, keepdims=True)
    i2 = jnp.min(jnp.where(p2 == m2, lane, E), axis=1, keepdims=True)

    oh = (lane == i1).astype(jnp.float32) + (lane == i2).astype(jnp.float32)
    base = jnp.zeros((1, E), jnp.float32)
    chunks = []
    CH = 512
    ri = lax.broadcasted_iota(jnp.int32, (CH, CH), 0)
    ci = lax.broadcasted_iota(jnp.int32, (CH, CH), 1)
    tril = (ci < ri).astype(jnp.float32)
    for c in range(T // CH):
        ohc = oh[c * CH:(c + 1) * CH, :]
        excl = jnp.dot(tril, ohc, preferred_element_type=jnp.float32) + base
        chunks.append(excl)
        base = base + jnp.sum(ohc, axis=0, keepdims=True)
    posm = jnp.concatenate(chunks, axis=0)
    counts = base
    countc = jnp.minimum(counts, float(CAP))

    nb_e = jnp.ceil(countc / BC)
    lane16 = lax.broadcasted_iota(jnp.int32, (1, E), 1)
    ue = lax.broadcasted_iota(jnp.int32, (E, E), 0)
    ve = lax.broadcasted_iota(jnp.int32, (E, E), 1)
    umat = (ue < ve).astype(jnp.float32)
    blk_start = jnp.dot(nb_e, umat, preferred_element_type=jnp.float32)
    nbl = jnp.sum(nb_e)
    row_off = BC * blk_start

    pos1 = jnp.sum(jnp.where(lane == i1, posm, 0.0), axis=1, keepdims=True)
    pos2 = jnp.sum(jnp.where(lane == i2, posm, 0.0), axis=1, keepdims=True)
    roff1 = jnp.sum(jnp.where(lane == i1, row_off, 0.0), axis=1, keepdims=True)
    roff2 = jnp.sum(jnp.where(lane == i2, row_off, 0.0), axis=1, keepdims=True)
    keep1 = pos1 < CAP
    keep2 = pos2 < CAP
    dst1 = jnp.where(keep1, roff1 + pos1, float(TRASH)).astype(jnp.int32)
    dst2 = jnp.where(keep2, roff2 + pos2, float(TRASH)).astype(jnp.int32)

    col2 = lax.broadcasted_iota(jnp.int32, (T, K), 1)
    dst_ref[...] = jnp.where(col2 == 0, dst1, dst2)

    g1 = jnp.where(keep1, m1, 0.0)
    g2 = jnp.where(keep2, m2, 0.0)
    colg = lax.broadcasted_iota(jnp.int32, (T, 2 * GW), 1)
    gw_ref[...] = jnp.where(colg < GW, g1, g2)

    bi = lax.broadcasted_iota(jnp.int32, (NBP, E), 0).astype(jnp.float32)
    in_blk = (bi >= blk_start) & (bi < blk_start + nb_e)
    lane_r = lax.broadcasted_iota(jnp.int32, (NBP, E), 1)
    be_raw = jnp.sum(jnp.where(in_blk, lane_r, 0), axis=1, keepdims=True)
    last_e = jnp.max(jnp.where(nb_e > 0, lane16, -1))
    bcol = lax.broadcasted_iota(jnp.int32, (NBP, 1), 0).astype(jnp.float32)
    be = jnp.where(bcol < nbl, be_raw, last_e)
    nbl_col = jnp.full((NBP, 1), nbl).astype(jnp.int32)
    meta_ref[...] = jnp.concatenate([be, nbl_col], axis=0)


_router = pl.pallas_call(
    _router_body,
    out_shape=(
        jax.ShapeDtypeStruct((T, K), jnp.int32),
        jax.ShapeDtypeStruct((T, 2 * GW), jnp.float32),
        jax.ShapeDtypeStruct((2 * NBP, 1), jnp.int32),
    ),
)


def _dispatch_body(x_hbm, dst_hbm, gw2_hbm, xin_hbm, gatec_hbm,
                   idxall_v, gwa_v, idx0, idx1, tok0, tok1, rows0, rows1,
                   gsem0, gsem1, ssem0, ssem1):
    wid = lax.axis_index("s") * NUM_SC_CORES + lax.axis_index("c")
    base = wid * APW

    pltpu.sync_copy(dst_hbm.at[pl.ds(base, APW)], idxall_v)
    pltpu.sync_copy(gw2_hbm.at[pl.ds(base, APW)], gwa_v)
    pltpu.sync_copy(gwa_v, gatec_hbm.at[idxall_v])

    idxb = (idx0, idx1)
    tokb = (tok0, tok1)
    rowsb = (rows0, rows1)
    gsem = (gsem0, gsem1)
    ssem = (ssem0, ssem1)
    gd = [None] * NCH
    sd = [None] * NCH

    def prep(c):
        b = c % 2
        a0 = base + c * ACH
        pltpu.sync_copy(dst_hbm.at[pl.ds(a0, ACH)], idxb[b])
        for j in range(ACH // 16):
            tokb[b][pl.ds(j * 16, 16)] = (a0 + j * 16 + lax.iota(jnp.int32, 16)) >> 1
        gd[c] = pltpu.async_copy(x_hbm.at[tokb[b]], rowsb[b], gsem[b])

    prep(0)
    prep(1)
    for c in range(NCH):
        b = c % 2
        gd[c].wait()
        sd[c] = pltpu.async_copy(rowsb[b], xin_hbm.at[idxb[b]], ssem[b])
        if 1 <= c < NCH - 1:
            sd[c - 1].wait()
            prep(c + 1)
    for c in range(max(0, NCH - 2), NCH):
        sd[c].wait()


@functools.lru_cache(maxsize=None)
def _get_dispatch():
    return pl.kernel(
        _dispatch_body,
        out_type=(
            jax.ShapeDtypeStruct((RT, H), jnp.float32),
            jax.ShapeDtypeStruct((RT, GW), jnp.float32),
        ),
        mesh=plsc.VectorSubcoreMesh(
            core_axis_name="c", subcore_axis_name="s",
            num_cores=NUM_SC_CORES, num_subcores=NUM_SC_SUBCORES),
        scratch_types=[
            pltpu.VMEM((APW,), jnp.int32),
            pltpu.VMEM((APW, GW), jnp.float32),
            pltpu.VMEM((ACH,), jnp.int32),
            pltpu.VMEM((ACH,), jnp.int32),
            pltpu.VMEM((ACH,), jnp.int32),
            pltpu.VMEM((ACH,), jnp.int32),
            pltpu.VMEM((ACH, H), jnp.float32),
            pltpu.VMEM((ACH, H), jnp.float32),
            pltpu.SemaphoreType.DMA,
            pltpu.SemaphoreType.DMA,
            pltpu.SemaphoreType.DMA,
            pltpu.SemaphoreType.DMA,
        ],
    )


def _ffn_body(meta_ref, xin_ref, w1_ref, w2_ref, gc_ref, out_ref):
    b = pl.program_id(0)
    nbl = meta_ref[NBP]

    @pl.when(b < nbl)
    def _():
        h = jnp.dot(xin_ref[...].astype(jnp.bfloat16),
                    w1_ref[...].astype(jnp.bfloat16),
                    preferred_element_type=jnp.float32)
        h = 0.5 * h * (1.0 + lax.erf(h * (2.0 ** -0.5)))
        acc = jnp.dot(h.astype(jnp.bfloat16),
                      w2_ref[...].astype(jnp.bfloat16),
                      preferred_element_type=jnp.float32)
        out_ref[...] = acc * gc_ref[...][:, 0:1]

    @pl.when(b == NB)
    def _():
        out_ref[...] = jnp.zeros_like(out_ref)


def _act_map(b, m):
    return jnp.minimum(b, m[NBP] - 1)


def _ffn_xin_map(b, m):
    return (_act_map(b, m), 0)


def _ffn_w1_map(b, m):
    return (0, m[b])


def _ffn_w2_map(b, m):
    return (m[b], 0)


def _ffn_gc_map(b, m):
    return (_act_map(b, m), 0)


def _ffn_out_map(b, m):
    return (jnp.where(b == NB, NB, _act_map(b, m)), 0)


_ffn = pl.pallas_call(
    _ffn_body,
    grid_spec=pltpu.PrefetchScalarGridSpec(
        num_scalar_prefetch=1,
        grid=(NBP,),
        in_specs=[
            pl.BlockSpec((BC, H), _ffn_xin_map),
            pl.BlockSpec((H, BF), _ffn_w1_map),
            pl.BlockSpec((BF, H), _ffn_w2_map),
            pl.BlockSpec((BC, GW), _ffn_gc_map),
        ],
        out_specs=pl.BlockSpec((BC, H), _ffn_out_map),
    ),
    out_shape=jax.ShapeDtypeStruct((RT, H), jnp.float32),
)


def _combine_body(yout_hbm, dst_hbm, out_hbm,
                  idx0, idx1, rows0, rows1, o0, o1,
                  gsem0, gsem1, wsem0, wsem1):
    wid = lax.axis_index("s") * NUM_SC_CORES + lax.axis_index("c")
    base_a = wid * APW
    base_t = wid * (APW // K)

    idxb = (idx0, idx1)
    rowsb = (rows0, rows1)
    ob = (o0, o1)
    gsem = (gsem0, gsem1)
    wsem = (wsem0, wsem1)
    gd = [None] * NCH
    wd = [None] * NCH

    def prep(c):
        b = c % 2
        a0 = base_a + c * ACH
        pltpu.sync_copy(dst_hbm.at[pl.ds(a0, ACH)], idxb[b])
        gd[c] = pltpu.async_copy(yout_hbm.at[idxb[b]], rowsb[b], gsem[b])

    prep(0)
    prep(1)
    for c in range(NCH):
        b = c % 2
        t0 = base_t + c * TCH
        gd[c].wait()
        if c >= 2:
            wd[c - 2].wait()
        rows = rowsb[b]
        o_v = ob[b]

        def tok_body(k, _):
            for cc in range(H // 16):
                r0 = rows[2 * k, pl.ds(cc * 16, 16)]
                r1 = rows[2 * k + 1, pl.ds(cc * 16, 16)]
                o_v[k, pl.ds(cc * 16, 16)] = r0 + r1
            return 0

        lax.fori_loop(0, TCH, tok_body, 0)
        wd[c] = pltpu.async_copy(o_v, out_hbm.at[pl.ds(t0, TCH)], wsem[b])
        if c + 2 < NCH:
            prep(c + 2)
    for c in range(max(0, NCH - 2), NCH):
        wd[c].wait()


@functools.lru_cache(maxsize=None)
def _get_combine():
    return pl.kernel(
        _combine_body,
        out_type=jax.ShapeDtypeStruct((T, H), jnp.float32),
        mesh=plsc.VectorSubcoreMesh(
            core_axis_name="c", subcore_axis_name="s",
            num_cores=NUM_SC_CORES, num_subcores=NUM_SC_SUBCORES),
        scratch_types=[
            pltpu.VMEM((ACH,), jnp.int32),
            pltpu.VMEM((ACH,), jnp.int32),
            pltpu.VMEM((ACH, H), jnp.float32),
            pltpu.VMEM((ACH, H), jnp.float32),
            pltpu.VMEM((TCH, H), jnp.float32),
            pltpu.VMEM((TCH, H), jnp.float32),
            pltpu.SemaphoreType.DMA,
            pltpu.SemaphoreType.DMA,
            pltpu.SemaphoreType.DMA,
            pltpu.SemaphoreType.DMA,
        ],
    )


def kernel(x, router_weight, w1, w2):
    dst2, gw, meta = _router(x, router_weight)
    dst = dst2.reshape(T * K)
    gw2 = gw.reshape(T * K, GW)
    meta_flat = meta.reshape(2 * NBP)
    xin, gatec = _get_dispatch()(x, dst, gw2)
    yout = _ffn(meta_flat, xin, w1, w2, gatec)
    out = _get_combine()(yout, dst)
    return out
